# baseline (speedup 1.0000x reference)
import jax
import jax.numpy as jnp
from jax import lax
from jax.experimental import pallas as pl
from jax.experimental.pallas import tpu as pltpu
from jax.experimental.pallas import tpu_sc as plsc

NC = 2
NS = 16
NW = NC * NS

B_ROWS = 16384
B_COLS = 200
D = 64
PACK_ROWS = B_ROWS * B_COLS // 2
PLANES_PER_W = B_ROWS // NW
K = 2
PACK_PER_CHUNK = K * B_COLS // 2
NCH = PLANES_PER_W // K
NPAIR = NCH // 2
GATHER_SPLITS = ((0, 128), (128, B_COLS - 128))


def _body(x_hbm, w_hbm, out_hbm, idx_v, rows_v, gsem0, gsem1, ssem0, ssem1, isem):
    c = lax.axis_index("c")
    s_ax = lax.axis_index("s")
    wid = s_ax * NC + c
    p0 = wid * PLANES_PER_W
    r0 = wid * PLANES_PER_W * B_COLS // 2

    gsems = (gsem0, gsem1)
    ssems = (ssem0, ssem1)

    def stage_idx(i, b, sync):
        src = x_hbm.at[pl.ds(p0 + i * K, K)]
        if sync:
            pltpu.sync_copy(src, idx_v.at[b])
        else:
            pltpu.async_copy(src, idx_v.at[b], isem)

    def wait_idx(b):
        pltpu.make_async_copy(
            x_hbm.at[pl.ds(p0, K)], idx_v.at[b], isem
        ).wait()

    def fire_gathers(b):
        for k in range(K):
            for off, n in GATHER_SPLITS:
                pltpu.async_copy(
                    w_hbm.at[idx_v.at[b, k, pl.ds(off, n)]],
                    rows_v.at[b, k, pl.ds(off, n)],
                    gsems[b],
                )

    def drain_gathers(b):
        for k in range(K):
            for off, n in GATHER_SPLITS:
                pltpu.make_async_copy(
                    w_hbm.at[idx_v.at[b, k, pl.ds(off, n)]],
                    rows_v.at[b, k, pl.ds(off, n)],
                    gsems[b],
                ).wait()

    def issue_store(i, b):
        pltpu.async_copy(
            rows_v.at[b],
            out_hbm.at[pl.ds(p0 + i * K, K)],
            ssems[b],
        )

    def drain_store(b):
        pltpu.make_async_copy(
            rows_v.at[b],
            out_hbm.at[pl.ds(p0, K)],
            ssems[b],
        ).wait()

    stage_idx(0, 0, sync=True)

    def pair(p, carry):
        i0 = 2 * p
        @pl.when(p > 0)
        def _():
            drain_store(0)
            wait_idx(0)

        fire_gathers(0)

        @pl.when(p > 0)
        def _():
            drain_gathers(1)
            issue_store(i0 - 1, 1)

        stage_idx(i0 + 1, 1, sync=False)

        @pl.when(p > 0)
        def _():
            drain_store(1)

        wait_idx(1)
        fire_gathers(1)
        drain_gathers(0)
        issue_store(i0, 0)

        @pl.when(p < NPAIR - 1)
        def _():
            stage_idx(i0 + 2, 0, sync=False)

        return carry

    lax.fori_loop(0, NPAIR, pair, 0)
    drain_gathers(1)
    issue_store(NCH - 1, 1)
    drain_store(0)
    drain_store(1)


def _tc_relayout(packed_ref, out_ref):
    nb = out_ref.shape[0]
    half = B_COLS // 2
    blk = packed_ref[...].reshape(nb, half, 2 * D)
    ev = blk[:, :, :D]
    od = blk[:, :, D:]
    out_ref[...] = jnp.stack([ev, od], axis=2).reshape(nb, B_COLS, D)


ROWS_BLK = 32


@jax.jit
def kernel(x, W):
    xi = x.astype(jnp.int32)
    mesh = plsc.VectorSubcoreMesh(
        core_axis_name="c", subcore_axis_name="s", num_cores=NC, num_subcores=NS
    )
    packed = pl.kernel(
        _body,
        out_type=jax.ShapeDtypeStruct((B_ROWS, B_COLS, D), jnp.float32),
        mesh=mesh,
        scratch_types=[
            pltpu.VMEM((2, K, B_COLS), jnp.int32),
            pltpu.VMEM((2, K, B_COLS, D), jnp.float32),
            pltpu.SemaphoreType.DMA,
            pltpu.SemaphoreType.DMA,
            pltpu.SemaphoreType.DMA,
            pltpu.SemaphoreType.DMA,
            pltpu.SemaphoreType.DMA,
        ],
        compiler_params=pltpu.CompilerParams(use_tc_tiling_on_sc=False),
    )(xi, W)
    packed = packed.reshape(PACK_ROWS, 2 * D)

    out = pl.pallas_call(
        _tc_relayout,
        grid=(B_ROWS // ROWS_BLK,),
        in_specs=[
            pl.BlockSpec(
                (ROWS_BLK * B_COLS // 2, 2 * D), lambda i: (i, 0)
            )
        ],
        out_specs=pl.BlockSpec((ROWS_BLK, B_COLS, D), lambda i: (i, 0, 0)),
        out_shape=jax.ShapeDtypeStruct((B_ROWS, B_COLS, D), jnp.float32),
    )(packed)
    return out


# speedup vs baseline: 1.8201x; 1.8201x over previous
import jax
import jax.numpy as jnp
from jax import lax
from jax.experimental import pallas as pl
from jax.experimental.pallas import tpu as pltpu
from jax.experimental.pallas import tpu_sc as plsc

NC = 2
NS = 16
NW = NC * NS

B_ROWS = 16384
B_COLS = 200
D = 64
PLANES_PER_W = B_ROWS // NW
K = 2
CHUNK = K * B_COLS
NCH = PLANES_PER_W // K
NPAIR = NCH // 2
GATHER_SPLITS = ((0, 128), (128, B_COLS - 128))


def _body(x_hbm, w_hbm, out_hbm, idx_v, rows_v, gsem0, gsem1, ssem0, ssem1, isem):
    c = lax.axis_index("c")
    s_ax = lax.axis_index("s")
    wid = s_ax * NC + c
    p0 = wid * PLANES_PER_W

    gsems = (gsem0, gsem1)
    ssems = (ssem0, ssem1)

    def stage_idx(i, b, sync):
        src = x_hbm.at[pl.ds(p0 + i * K, K)]
        if sync:
            pltpu.sync_copy(src, idx_v.at[b])
        else:
            pltpu.async_copy(src, idx_v.at[b], isem)

    def wait_idx(b):
        pltpu.make_async_copy(
            x_hbm.at[pl.ds(p0, K)], idx_v.at[b], isem
        ).wait()

    def fire_gathers(b):
        for k in range(K):
            for off, n in GATHER_SPLITS:
                pltpu.async_copy(
                    w_hbm.at[idx_v.at[b, k, pl.ds(off, n)]],
                    rows_v.at[b, k, pl.ds(off, n)],
                    gsems[b],
                )

    def drain_gathers(b):
        for k in range(K):
            for off, n in GATHER_SPLITS:
                pltpu.make_async_copy(
                    w_hbm.at[idx_v.at[b, k, pl.ds(off, n)]],
                    rows_v.at[b, k, pl.ds(off, n)],
                    gsems[b],
                ).wait()

    def issue_store(i, b):
        pltpu.async_copy(
            rows_v.at[b], out_hbm.at[pl.ds(p0 + i * K, K)], ssems[b]
        )

    def drain_store(b):
        pltpu.make_async_copy(
            rows_v.at[b], out_hbm.at[pl.ds(p0, K)], ssems[b]
        ).wait()

    stage_idx(0, 0, sync=True)

    def pair(p, carry):
        i0 = 2 * p
        @pl.when(p > 0)
        def _():
            drain_store(0)
            wait_idx(0)

        fire_gathers(0)

        @pl.when(p > 0)
        def _():
            drain_gathers(1)
            issue_store(i0 - 1, 1)

        stage_idx(i0 + 1, 1, sync=False)

        @pl.when(p > 0)
        def _():
            drain_store(1)

        wait_idx(1)
        fire_gathers(1)
        drain_gathers(0)
        issue_store(i0, 0)

        @pl.when(p < NPAIR - 1)
        def _():
            stage_idx(i0 + 2, 0, sync=False)

        return carry

    lax.fori_loop(0, NPAIR, pair, 0)
    drain_gathers(1)
    issue_store(NCH - 1, 1)
    drain_store(0)
    drain_store(1)


@jax.jit
def kernel(x, W):
    xi = x.astype(jnp.int32)
    mesh = plsc.VectorSubcoreMesh(
        core_axis_name="c", subcore_axis_name="s", num_cores=NC, num_subcores=NS
    )
    w128 = jnp.pad(W, ((0, 0), (0, D)))
    out = pl.kernel(
        _body,
        out_type=jax.ShapeDtypeStruct((B_ROWS, B_COLS, 2 * D), jnp.float32),
        mesh=mesh,
        scratch_types=[
            pltpu.VMEM((2, K, B_COLS), jnp.int32),
            pltpu.VMEM((2, K, B_COLS, 2 * D), jnp.float32),
            pltpu.SemaphoreType.DMA,
            pltpu.SemaphoreType.DMA,
            pltpu.SemaphoreType.DMA,
            pltpu.SemaphoreType.DMA,
            pltpu.SemaphoreType.DMA,
        ],
        compiler_params=pltpu.CompilerParams(use_tc_tiling_on_sc=False),
    )(xi, w128)
    return out[:, :, :D]


# speedup vs baseline: 2.5854x; 1.4204x over previous
import jax
import jax.numpy as jnp
from jax import lax
from jax.experimental import pallas as pl
from jax.experimental.pallas import tpu as pltpu
from jax.experimental.pallas import tpu_sc as plsc

NC = 2
NS = 16
NW = NC * NS

B_ROWS = 16384
B_COLS = 200
D = 64
PLANES_PER_W = B_ROWS // NW
K = 2
CHUNK = K * B_COLS
NCH = PLANES_PER_W // K
NPAIR = NCH // 2
GATHER_SPLITS = ((0, 128), (128, B_COLS - 128))


def _body(x_hbm, w_hbm, out_hbm, idx_v, rows_v, gsem0, gsem1, ssem0, ssem1, isem):
    c = lax.axis_index("c")
    s_ax = lax.axis_index("s")
    wid = s_ax * NC + c
    p0 = wid * PLANES_PER_W

    gsems = (gsem0, gsem1)
    ssems = (ssem0, ssem1)

    def stage_idx(i, b, sync):
        src = x_hbm.at[pl.ds(p0 + i * K, K)]
        if sync:
            pltpu.sync_copy(src, idx_v.at[b])
        else:
            pltpu.async_copy(src, idx_v.at[b], isem)

    def wait_idx(b):
        pltpu.make_async_copy(
            x_hbm.at[pl.ds(p0, K)], idx_v.at[b], isem
        ).wait()

    def fire_gathers(b):
        for k in range(K):
            for off, n in GATHER_SPLITS:
                pltpu.async_copy(
                    w_hbm.at[idx_v.at[b, k, pl.ds(off, n)]],
                    rows_v.at[b, k, pl.ds(off, n)],
                    gsems[b],
                )

    def drain_gathers(b):
        for k in range(K):
            for off, n in GATHER_SPLITS:
                pltpu.make_async_copy(
                    w_hbm.at[idx_v.at[b, k, pl.ds(off, n)]],
                    rows_v.at[b, k, pl.ds(off, n)],
                    gsems[b],
                ).wait()

    def issue_store(i, b):
        pltpu.async_copy(
            rows_v.at[b],
            out_hbm.at[pl.ds(p0 + i * K, K), slice(None), pl.ds(0, D)],
            ssems[b],
        )

    def drain_store(b):
        pltpu.make_async_copy(
            rows_v.at[b],
            out_hbm.at[pl.ds(p0, K), slice(None), pl.ds(0, D)],
            ssems[b],
        ).wait()

    stage_idx(0, 0, sync=True)

    def pair(p, carry):
        i0 = 2 * p
        @pl.when(p > 0)
        def _():
            drain_store(0)
            wait_idx(0)

        fire_gathers(0)

        @pl.when(p > 0)
        def _():
            drain_gathers(1)
            issue_store(i0 - 1, 1)

        stage_idx(i0 + 1, 1, sync=False)

        @pl.when(p > 0)
        def _():
            drain_store(1)

        wait_idx(1)
        fire_gathers(1)
        drain_gathers(0)
        issue_store(i0, 0)

        @pl.when(p < NPAIR - 1)
        def _():
            stage_idx(i0 + 2, 0, sync=False)

        return carry

    lax.fori_loop(0, NPAIR, pair, 0)
    drain_gathers(1)
    issue_store(NCH - 1, 1)
    drain_store(0)
    drain_store(1)


@jax.jit
def kernel(x, W):
    xi = x.astype(jnp.int32)
    mesh = plsc.VectorSubcoreMesh(
        core_axis_name="c", subcore_axis_name="s", num_cores=NC, num_subcores=NS
    )
    out = pl.kernel(
        _body,
        out_type=jax.ShapeDtypeStruct((B_ROWS, B_COLS, 2 * D), jnp.float32),
        mesh=mesh,
        scratch_types=[
            pltpu.VMEM((2, K, B_COLS), jnp.int32),
            pltpu.VMEM((2, K, B_COLS, D), jnp.float32),
            pltpu.SemaphoreType.DMA,
            pltpu.SemaphoreType.DMA,
            pltpu.SemaphoreType.DMA,
            pltpu.SemaphoreType.DMA,
            pltpu.SemaphoreType.DMA,
        ],
        compiler_params=pltpu.CompilerParams(use_tc_tiling_on_sc=False),
    )(xi, W)
    return out[:, :, :D]


# speedup vs baseline: 2.5897x; 1.0017x over previous
import jax
import jax.numpy as jnp
from jax import lax
from jax.experimental import pallas as pl
from jax.experimental.pallas import tpu as pltpu
from jax.experimental.pallas import tpu_sc as plsc

NC = 2
NS = 16
NW = NC * NS

B_ROWS = 16384
B_COLS = 200
D = 64
PLANES_PER_W = B_ROWS // NW
K = 4
CHUNK = K * B_COLS
NCH = PLANES_PER_W // K
NPAIR = NCH // 2
GATHER_SPLITS = ((0, 128), (128, B_COLS - 128))


def _body(x_hbm, w_hbm, out_hbm, idx_v, rows_v, gsem0, gsem1, ssem0, ssem1, isem):
    c = lax.axis_index("c")
    s_ax = lax.axis_index("s")
    wid = s_ax * NC + c
    p0 = wid * PLANES_PER_W

    gsems = (gsem0, gsem1)
    ssems = (ssem0, ssem1)

    def stage_idx(i, b, sync):
        src = x_hbm.at[pl.ds(p0 + i * K, K)]
        if sync:
            pltpu.sync_copy(src, idx_v.at[b])
        else:
            pltpu.async_copy(src, idx_v.at[b], isem)

    def wait_idx(b):
        pltpu.make_async_copy(
            x_hbm.at[pl.ds(p0, K)], idx_v.at[b], isem
        ).wait()

    def fire_gathers(b):
        for k in range(K):
            for off, n in GATHER_SPLITS:
                pltpu.async_copy(
                    w_hbm.at[idx_v.at[b, k, pl.ds(off, n)]],
                    rows_v.at[b, k, pl.ds(off, n)],
                    gsems[b],
                )

    def drain_gathers(b):
        for k in range(K):
            for off, n in GATHER_SPLITS:
                pltpu.make_async_copy(
                    w_hbm.at[idx_v.at[b, k, pl.ds(off, n)]],
                    rows_v.at[b, k, pl.ds(off, n)],
                    gsems[b],
                ).wait()

    def issue_store(i, b):
        pltpu.async_copy(
            rows_v.at[b],
            out_hbm.at[pl.ds(p0 + i * K, K), slice(None), pl.ds(0, D)],
            ssems[b],
        )

    def drain_store(b):
        pltpu.make_async_copy(
            rows_v.at[b],
            out_hbm.at[pl.ds(p0, K), slice(None), pl.ds(0, D)],
            ssems[b],
        ).wait()

    stage_idx(0, 0, sync=True)

    def pair(p, carry):
        i0 = 2 * p
        @pl.when(p > 0)
        def _():
            drain_store(0)
            wait_idx(0)

        fire_gathers(0)

        @pl.when(p > 0)
        def _():
            drain_gathers(1)
            issue_store(i0 - 1, 1)

        stage_idx(i0 + 1, 1, sync=False)

        @pl.when(p > 0)
        def _():
            drain_store(1)

        wait_idx(1)
        fire_gathers(1)
        drain_gathers(0)
        issue_store(i0, 0)

        @pl.when(p < NPAIR - 1)
        def _():
            stage_idx(i0 + 2, 0, sync=False)

        return carry

    lax.fori_loop(0, NPAIR, pair, 0)
    drain_gathers(1)
    issue_store(NCH - 1, 1)
    drain_store(0)
    drain_store(1)


@jax.jit
def kernel(x, W):
    xi = x.astype(jnp.int32)
    mesh = plsc.VectorSubcoreMesh(
        core_axis_name="c", subcore_axis_name="s", num_cores=NC, num_subcores=NS
    )
    out = pl.kernel(
        _body,
        out_type=jax.ShapeDtypeStruct((B_ROWS, B_COLS, 2 * D), jnp.float32),
        mesh=mesh,
        scratch_types=[
            pltpu.VMEM((2, K, B_COLS), jnp.int32),
            pltpu.VMEM((2, K, B_COLS, D), jnp.float32),
            pltpu.SemaphoreType.DMA,
            pltpu.SemaphoreType.DMA,
            pltpu.SemaphoreType.DMA,
            pltpu.SemaphoreType.DMA,
            pltpu.SemaphoreType.DMA,
        ],
        compiler_params=pltpu.CompilerParams(use_tc_tiling_on_sc=False),
    )(xi, W)
    return out[:, :, :D]
